# Initial kernel scaffold; baseline (speedup 1.0000x reference)
import jax
import jax.numpy as jnp
from jax.experimental import pallas as pl


def kernel(x, edge_index, W_msg, W_film, b_film):
    raise NotImplementedError("write your pallas kernel here")


# speedup vs baseline: 61.9188x; 61.9188x over previous
import functools

import jax
import jax.numpy as jnp
from jax import lax
from jax.experimental import pallas as pl
from jax.experimental.pallas import tpu as pltpu
from jax.experimental.pallas import tpu_sc as plsc

NC = 2
NS = 16
NW = NC * NS
K = 1024
BR = 2000


def _mesh():
    return plsc.VectorSubcoreMesh(
        core_axis_name="c", subcore_axis_name="s", num_cores=NC, num_subcores=NS)


def _zero_acc(rows, acc, zbase, rpt, D):
    @pl.loop(0, K)
    def _(i):
        rows[i, :] = jnp.zeros((16,), jnp.float32)
    nfull, rem = rpt // K, rpt % K
    for j in range(nfull):
        pltpu.sync_copy(rows, acc.at[pl.ds(zbase + j * K, K)])
    if rem:
        pltpu.sync_copy(rows.at[pl.ds(0, rem)],
                        acc.at[pl.ds(zbase + nfull * K, rem)])


def _write_out(acc, out_hbm, obase, zbase, rpt):
    nfull, rem = rpt // K, rpt % K
    for j in range(nfull):
        pltpu.sync_copy(acc.at[pl.ds(zbase + j * K, K)],
                        out_hbm.at[pl.ds(obase + j * K, K)])
    if rem:
        pltpu.sync_copy(acc.at[pl.ds(zbase + nfull * K, rem)],
                        out_hbm.at[pl.ds(obase + nfull * K, rem)])


def _make_edge_kernel(A, D, n_chunks):
    rpt = A // NS

    @functools.partial(
        pl.kernel,
        out_type=jax.ShapeDtypeStruct((NC * A, D), jnp.float32),
        mesh=_mesh(),
        compiler_params=pltpu.CompilerParams(use_tc_tiling_on_sc=False),
        scratch_types=[
            pltpu.VMEM((K,), jnp.int32),
            pltpu.VMEM((K,), jnp.int32),
            pltpu.VMEM((K, D), jnp.float32),
            pltpu.VMEM_SHARED((A, D), jnp.float32),
            pltpu.SemaphoreType.DMA,
        ],
    )
    def k(proj_hbm, src_hbm, dst_hbm, out_hbm, sidx, didx, rows, acc, sem):
        c = lax.axis_index("c")
        s = lax.axis_index("s")
        wid = s * NC + c
        zbase = s * rpt
        _zero_acc(rows, acc, zbase, rpt, D)
        plsc.subcore_barrier()
        ebase = wid * (n_chunks * K)

        @pl.loop(0, n_chunks)
        def _(i):
            off = ebase + i * K
            pltpu.sync_copy(src_hbm.at[pl.ds(off, K)], sidx)
            pltpu.sync_copy(dst_hbm.at[pl.ds(off, K)], didx)
            pltpu.async_copy(proj_hbm.at[sidx], rows, sem).wait()
            pltpu.sync_copy(rows, acc.at[didx], add=True)

        plsc.subcore_barrier()
        _write_out(acc, out_hbm, c * A + zbase, zbase, rpt)

    return k


def _make_deg_kernel(A, D, n_chunks):
    rpt = A // NS

    @functools.partial(
        pl.kernel,
        out_type=jax.ShapeDtypeStruct((NC * A, D), jnp.float32),
        mesh=_mesh(),
        compiler_params=pltpu.CompilerParams(use_tc_tiling_on_sc=False),
        scratch_types=[
            pltpu.VMEM((K,), jnp.int32),
            pltpu.VMEM((K, D), jnp.float32),
            pltpu.VMEM_SHARED((A, D), jnp.float32),
        ],
    )
    def k(dst_hbm, out_hbm, didx, rows, acc):
        c = lax.axis_index("c")
        s = lax.axis_index("s")
        wid = s * NC + c
        zbase = s * rpt
        _zero_acc(rows, acc, zbase, rpt, D)
        plsc.subcore_barrier()

        @pl.loop(0, K)
        def _(i):
            rows[i, :] = jnp.ones((16,), jnp.float32)

        ebase = wid * (n_chunks * K)

        @pl.loop(0, n_chunks)
        def _(i):
            off = ebase + i * K
            pltpu.sync_copy(dst_hbm.at[pl.ds(off, K)], didx)
            pltpu.sync_copy(rows, acc.at[didx], add=True)

        plsc.subcore_barrier()
        _write_out(acc, out_hbm, c * A + zbase, zbase, rpt)

    return k


def _mm0_body(x_ref, wm_ref, wf_ref, b_ref, proj_ref, film_ref):
    h = x_ref[...]
    proj_ref[...] = jnp.dot(h, wm_ref[...], preferred_element_type=jnp.float32)
    film_ref[...] = (jnp.dot(h, wf_ref[...], preferred_element_type=jnp.float32)
                     + b_ref[0:1, :])


def _combine(s0_ref, s1_ref, f_ref, d0_ref, d1_ref, D):
    Ssum = s0_ref[0] + s1_ref[0]
    deg = d0_ref[0] + d1_ref[0]
    gamma = f_ref[:, :D]
    beta = f_ref[:, D:]
    return jnp.maximum(gamma * Ssum + deg * beta, 0.0)


def _make_mm0(N, D):
    grid = N // BR
    return pl.pallas_call(
        _mm0_body,
        grid=(grid,),
        in_specs=[
            pl.BlockSpec((BR, D), lambda i: (i, 0)),
            pl.BlockSpec((D, D), lambda i: (0, 0)),
            pl.BlockSpec((D, 2 * D), lambda i: (0, 0)),
            pl.BlockSpec((8, 2 * D), lambda i: (0, 0)),
        ],
        out_specs=[
            pl.BlockSpec((BR, D), lambda i: (i, 0)),
            pl.BlockSpec((BR, 2 * D), lambda i: (i, 0)),
        ],
        out_shape=[
            jax.ShapeDtypeStruct((N, D), jnp.float32),
            jax.ShapeDtypeStruct((N, 2 * D), jnp.float32),
        ],
    )


def _make_mmc(N, A, D):
    grid = N // BR
    nblk_a = A // BR

    def body(s_ref, s1_ref, f_ref, d0_ref, d1_ref, wm_ref, wf_ref, b_ref,
             proj_ref, film_ref):
        h = _combine(s_ref, s1_ref, f_ref, d0_ref, d1_ref, D)
        proj_ref[...] = jnp.dot(h, wm_ref[...],
                                preferred_element_type=jnp.float32)
        film_ref[...] = (jnp.dot(h, wf_ref[...],
                                 preferred_element_type=jnp.float32)
                         + b_ref[0:1, :])

    sp = pl.BlockSpec((1, BR, D), lambda i: (0, i, 0))
    return pl.pallas_call(
        body,
        grid=(grid,),
        in_specs=[
            sp,
            pl.BlockSpec((1, BR, D), lambda i: (1, i, 0)),
            pl.BlockSpec((BR, 2 * D), lambda i: (i, 0)),
            sp,
            pl.BlockSpec((1, BR, D), lambda i: (1, i, 0)),
            pl.BlockSpec((D, D), lambda i: (0, 0)),
            pl.BlockSpec((D, 2 * D), lambda i: (0, 0)),
            pl.BlockSpec((8, 2 * D), lambda i: (0, 0)),
        ],
        out_specs=[
            pl.BlockSpec((BR, D), lambda i: (i, 0)),
            pl.BlockSpec((BR, 2 * D), lambda i: (i, 0)),
        ],
        out_shape=[
            jax.ShapeDtypeStruct((N, D), jnp.float32),
            jax.ShapeDtypeStruct((N, 2 * D), jnp.float32),
        ],
    )


def _make_mmf(N, A, D):
    grid = N // BR

    def body(s_ref, s1_ref, f_ref, d0_ref, d1_ref, h_ref):
        h_ref[...] = _combine(s_ref, s1_ref, f_ref, d0_ref, d1_ref, D)

    return pl.pallas_call(
        body,
        grid=(grid,),
        in_specs=[
            pl.BlockSpec((1, BR, D), lambda i: (0, i, 0)),
            pl.BlockSpec((1, BR, D), lambda i: (1, i, 0)),
            pl.BlockSpec((BR, 2 * D), lambda i: (i, 0)),
            pl.BlockSpec((1, BR, D), lambda i: (0, i, 0)),
            pl.BlockSpec((1, BR, D), lambda i: (1, i, 0)),
        ],
        out_specs=[pl.BlockSpec((BR, D), lambda i: (i, 0))],
        out_shape=[jax.ShapeDtypeStruct((N, D), jnp.float32)],
    )


def kernel(x, edge_index, W_msg, W_film, b_film):
    N, D = x.shape
    E = edge_index.shape[1]
    L = W_msg.shape[0]
    assert D == 16

    A = -(-(N + 1) // NS) * NS
    n_chunks = -(-E // (NW * K))
    E_pad = NW * K * n_chunks
    pad = E_pad - E

    src = edge_index[0]
    dst = edge_index[1]
    if pad:
        src = jnp.concatenate([src, jnp.zeros((pad,), jnp.int32)])
        dst = jnp.concatenate([dst, jnp.full((pad,), N, jnp.int32)])

    edge_k = _make_edge_kernel(A, D, n_chunks)
    deg_k = _make_deg_kernel(A, D, n_chunks)
    mm0 = _make_mm0(N, D)
    mmc = _make_mmc(N, A, D)
    mmf = _make_mmf(N, A, D)

    b2 = jnp.broadcast_to(b_film[:, None, :], (L, 8, 2 * D))

    degp = deg_k(dst).reshape(NC, A, D)
    proj, film = mm0(x, W_msg[0], W_film[0], b2[0])
    for l in range(L):
        Sp = edge_k(proj, src, dst).reshape(NC, A, D)
        if l < L - 1:
            proj, film = mmc(Sp, Sp, film, degp, degp,
                             W_msg[l + 1], W_film[l + 1], b2[l + 1])
        else:
            (h,) = mmf(Sp, Sp, film, degp, degp)
    return h
